# baseline (speedup 1.0000x reference)
import functools

import jax
import jax.numpy as jnp
from jax import lax
from jax.experimental import pallas as pl
from jax.experimental.pallas import tpu as pltpu
from jax.experimental.pallas import tpu_sc as plsc

_B, _TG = 64, 8192
_V, _D = 4096, 64
_C_NO, _TOP_K = 3500, 1024
_CHUNK = 128
_NCHUNK = _TG // _CHUNK
_NPAIR = _NCHUNK // 2
_INT_MIN = -2147483648


def _score_body(body_hbm, ent_hbm, rel_hbm, tail_hbm, head_hbm,
                g_hbm, pt_hbm, ph_hbm, p_hbm, a1_hbm, a2_hbm,
                p_v, a1_v, a2_v, tidx_v, hidx_v, g_v, pt_v, ph_v,
                e1a_v, e2a_v, rra_v, e1b_v, e2b_v, rrb_v, t_v,
                raw_a, raw_b, sem_pg, sem_gr, sem_raw):
    cid = lax.axis_index("c")
    sid = lax.axis_index("s")
    wid = sid * 2 + cid

    def ground_start(ci, bufs):
        e1, e2, rr = bufs
        sl = pl.ds(ci * _CHUNK, _CHUNK)
        pltpu.async_copy(ent_hbm.at[a1_v.at[sl]], e1, sem_gr)
        pltpu.async_copy(ent_hbm.at[a2_v.at[sl]], e2, sem_gr)
        pltpu.async_copy(rel_hbm.at[p_v.at[sl]], rr, sem_gr)

    def ground_wait(bufs):
        e1, e2, rr = bufs
        pltpu.make_async_copy(ent_hbm.at[a1_v.at[pl.ds(0, _CHUNK)]], e1, sem_gr).wait()
        pltpu.make_async_copy(ent_hbm.at[a2_v.at[pl.ds(0, _CHUNK)]], e2, sem_gr).wait()
        pltpu.make_async_copy(rel_hbm.at[p_v.at[pl.ds(0, _CHUNK)]], rr, sem_gr).wait()

    def pg_start(ci):
        sl = pl.ds(ci * _CHUNK, _CHUNK)
        pltpu.async_copy(tail_hbm.at[tidx_v.at[sl]], pt_v.at[sl], sem_pg)
        pltpu.async_copy(head_hbm.at[hidx_v.at[sl]], ph_v.at[sl], sem_pg)

    def pg_drain(ci):
        sl = pl.ds(ci * _CHUNK, _CHUNK)
        pltpu.make_async_copy(tail_hbm.at[tidx_v.at[sl]], pt_v.at[sl], sem_pg).wait()
        pltpu.make_async_copy(head_hbm.at[hidx_v.at[sl]], ph_v.at[sl], sem_pg).wait()

    def ground_compute(ci, bufs):
        e1, e2, rr = bufs

        def grp_body(gi, _):
            for e16 in range(16):
                e = gi * 16 + e16
                part = (e1[e, pl.ds(0, 16)] * e2[e, pl.ds(0, 16)]
                        * rr[e, pl.ds(0, 16)])
                for j in (16, 32, 48):
                    part = part + (e1[e, pl.ds(j, 16)]
                                   * e2[e, pl.ds(j, 16)]
                                   * rr[e, pl.ds(j, 16)])
                t_v[pl.ds(e16 * 16, 16)] = part
            lanes = lax.iota(jnp.int32, 16) * 16
            acc = plsc.load_gather(t_v, [lanes])
            for c in range(1, 16):
                acc = acc + plsc.load_gather(t_v, [lanes + c])
            g_v[pl.ds(ci * _CHUNK + gi * 16, 16)] = acc
            return 0
        lax.fori_loop(0, _CHUNK // 16, grp_body, 0)

    bufs_a = None

    _RAWC = 512
    _RAWW = _RAWC * 12

    def extract(q, buf):
        def ext_body(i, _):
            sl16 = pl.ds(q * _RAWC + i * 16, 16)
            lanes = (lax.iota(jnp.int32, 16) + i * 16) * 12
            p_v[sl16] = plsc.load_gather(buf, [lanes])
            a1_v[sl16] = plsc.load_gather(buf, [lanes + 1])
            a2_v[sl16] = plsc.load_gather(buf, [lanes + 2])
            return 0
        lax.fori_loop(0, _RAWC // 16, ext_body, 0)

    for r in range(2):
        row = wid * 2 + r
        nq = _TG // _RAWC
        cur = pltpu.async_copy(
            body_hbm.at[row, pl.ds(0, _RAWW)], raw_a, sem_raw)
        for q in range(nq):
            nxt = None
            if q < nq - 1:
                nxt = pltpu.async_copy(
                    body_hbm.at[row, pl.ds((q + 1) * _RAWW, _RAWW)],
                    raw_b if q % 2 == 0 else raw_a, sem_raw)
            cur.wait()
            extract(q, raw_a if q % 2 == 0 else raw_b)
            cur = nxt

        def idx_body(i, _):
            sl = pl.ds(i * 16, 16)
            base = p_v[sl] * _V
            tidx_v[sl] = base + a1_v[sl]
            hidx_v[sl] = base + a2_v[sl]
            return 0
        lax.fori_loop(0, _TG // 16, idx_body, 0)

        A = (e1a_v, e2a_v, rra_v)
        Bb = (e1b_v, e2b_v, rrb_v)

        ground_start(0, A)

        def pair_body(cj, _):
            c0 = cj * 2
            ground_start(c0 + 1, Bb)
            pg_start(c0)
            pg_start(c0 + 1)
            ground_wait(A)
            ground_compute(c0, A)

            @pl.when(cj > 0)
            def _():
                pg_drain(c0 - 2)
                pg_drain(c0 - 1)

            @pl.when(cj < _NPAIR - 1)
            def _():
                ground_start(c0 + 2, A)

            ground_wait(Bb)
            ground_compute(c0 + 1, Bb)
            return 0
        lax.fori_loop(0, _NPAIR, pair_body, 0)

        pg_drain(_NCHUNK - 2)
        pg_drain(_NCHUNK - 1)

        pltpu.sync_copy(g_v, g_hbm.at[row])
        pltpu.sync_copy(pt_v, pt_hbm.at[row])
        pltpu.sync_copy(ph_v, ph_hbm.at[row])
        pltpu.sync_copy(p_v, p_hbm.at[row])
        pltpu.sync_copy(a1_v, a1_hbm.at[row])
        pltpu.sync_copy(a2_v, a2_hbm.at[row])


_score_call = functools.partial(
    pl.kernel,
    out_type=(
        jax.ShapeDtypeStruct((_B, _TG), jnp.float32),
        jax.ShapeDtypeStruct((_B, _TG), jnp.float32),
        jax.ShapeDtypeStruct((_B, _TG), jnp.float32),
        jax.ShapeDtypeStruct((_B, _TG), jnp.int32),
        jax.ShapeDtypeStruct((_B, _TG), jnp.int32),
        jax.ShapeDtypeStruct((_B, _TG), jnp.int32),
    ),
    mesh=plsc.VectorSubcoreMesh(core_axis_name="c", subcore_axis_name="s"),
    compiler_params=pltpu.CompilerParams(
        needs_layout_passes=False, use_tc_tiling_on_sc=False),
    scratch_types=[
        pltpu.VMEM((_TG,), jnp.int32),
        pltpu.VMEM((_TG,), jnp.int32),
        pltpu.VMEM((_TG,), jnp.int32),
        pltpu.VMEM((_TG,), jnp.int32),
        pltpu.VMEM((_TG,), jnp.int32),
        pltpu.VMEM((_TG,), jnp.float32),
        pltpu.VMEM((_TG,), jnp.float32),
        pltpu.VMEM((_TG,), jnp.float32),
        pltpu.VMEM((_CHUNK, _D), jnp.float32),
        pltpu.VMEM((_CHUNK, _D), jnp.float32),
        pltpu.VMEM((_CHUNK, _D), jnp.float32),
        pltpu.VMEM((_CHUNK, _D), jnp.float32),
        pltpu.VMEM((_CHUNK, _D), jnp.float32),
        pltpu.VMEM((_CHUNK, _D), jnp.float32),
        pltpu.VMEM((256,), jnp.float32),
        pltpu.VMEM((512 * 12,), jnp.int32),
        pltpu.VMEM((512 * 12,), jnp.int32),
        pltpu.SemaphoreType.DMA,
        pltpu.SemaphoreType.DMA,
        pltpu.SemaphoreType.DMA,
    ],
)(_score_body)


def _select_body(p_ref, a1_ref, a2_ref, mask_ref, g_ref, pt_ref, ph_ref, out_ref):
    p = p_ref[...]
    a1 = a1_ref[...]
    a2 = a2_ref[...]
    mask = mask_ref[...] != 0
    g = g_ref[...]
    pt = pt_ref[...]
    ph = ph_ref[...]

    is_ground = mask & (a1 <= _C_NO) & (a2 <= _C_NO) & (p != 0)
    tail_case = (a1 > 0) & (a1 <= _C_NO) & (a2 > _C_NO)
    head_case = (a1 > _C_NO) & (a2 > 0) & (a2 <= _C_NO)
    is_partial = mask & (~is_ground) & (p != 0) & (tail_case | head_case)
    p_scores = jnp.where(tail_case, pt, jnp.where(head_case, ph, 0.0))
    scores = jnp.where(is_partial, p_scores, jnp.where(is_ground, g, 0.0))
    scored = is_ground | (is_partial & (p_scores > 0.0))
    uncond = mask & (~is_ground) & (~is_partial)

    topk = jnp.where(scored, scores, -jnp.inf)
    x = lax.bitcast_convert_type(topk, jnp.int32)
    s = x ^ ((x >> 31) & jnp.int32(0x7FFFFFFF))

    def step(i, P):
        b = jnp.int32(31) - i
        Pp = P | (jnp.int32(1) << b)
        v = Pp ^ jnp.int32(_INT_MIN)
        cnt = jnp.sum((s >= v).astype(jnp.int32), axis=1, keepdims=True)
        return jnp.where(cnt >= _TOP_K, Pp, P)

    P = lax.fori_loop(0, 32, step, jnp.zeros((_B, 1), jnp.int32))
    T = P ^ jnp.int32(_INT_MIN)

    gt = s > T
    cnt_gt = jnp.sum(gt.astype(jnp.int32), axis=1, keepdims=True)
    r = _TOP_K - cnt_gt
    eq = s == T
    inc = eq.astype(jnp.int32)
    sh = 1
    while sh < _TG:
        inc = inc + jnp.concatenate(
            [jnp.zeros((_B, sh), jnp.int32), inc[:, :_TG - sh]], axis=1)
        sh *= 2
    keep = gt | (eq & (inc <= r))
    new_mask = mask & (keep | uncond)
    out_ref[...] = new_mask.astype(jnp.int32)


def _select_call(p, a1, a2, mask_i32, g, pt, ph):
    return pl.pallas_call(
        _select_body,
        out_shape=jax.ShapeDtypeStruct((_B, _TG), jnp.int32),
    )(p, a1, a2, mask_i32, g, pt, ph)


def kernel(body, mask, rule_idx, d, ent_emb, rel_emb, max_tail_score, max_head_score):
    mask_i32 = mask.astype(jnp.int32)
    tail_flat = max_tail_score.reshape(-1)
    head_flat = max_head_score.reshape(-1)
    body_flat = body.reshape(_B, _TG * 12)
    g, pt, ph, p, a1, a2 = _score_call(
        body_flat, ent_emb, rel_emb, tail_flat, head_flat)
    keep = _select_call(p, a1, a2, mask_i32, g, pt, ph)
    new_mask = keep != 0
    return (body, new_mask, rule_idx)


# speedup vs baseline: 1.2155x; 1.2155x over previous
import functools

import jax
import jax.numpy as jnp
from jax import lax
from jax.experimental import pallas as pl
from jax.experimental.pallas import tpu as pltpu
from jax.experimental.pallas import tpu_sc as plsc

_B, _TG = 64, 8192
_V, _D = 4096, 64
_C_NO, _TOP_K = 3500, 1024
_GC = 64
_QC = 2048
_PC = 128
_INT_MIN = -2147483648


def _score_body(first_hbm, mask_hbm, ent_hbm, rel_hbm, tail_hbm, head_hbm,
                g_hbm, pt_hbm, ph_hbm,
                pq_v, a1q_v, a2q_v, mq_v, tidx_v, hidx_v,
                ca1_v, ca2_v, cp_v, ceid_v, g_v, pt_v, ph_v,
                e1a_v, e2a_v, rra_v, e1b_v, e2b_v, rrb_v, t_v,
                sem_pg, sem_gr, sem_q):
    cid = lax.axis_index("c")
    sid = lax.axis_index("s")
    wid = sid * 2 + cid

    iota = lax.iota(jnp.int32, 16)

    def ground_start(ci, bufs):
        e1, e2, rr = bufs
        sl = pl.ds(ci * _GC, _GC)
        pltpu.async_copy(ent_hbm.at[ca1_v.at[sl]], e1, sem_gr)
        pltpu.async_copy(ent_hbm.at[ca2_v.at[sl]], e2, sem_gr)
        pltpu.async_copy(rel_hbm.at[cp_v.at[sl]], rr, sem_gr)

    def ground_wait(bufs):
        e1, e2, rr = bufs
        sl = pl.ds(0, _GC)
        pltpu.make_async_copy(ent_hbm.at[ca1_v.at[sl]], e1, sem_gr).wait()
        pltpu.make_async_copy(ent_hbm.at[ca2_v.at[sl]], e2, sem_gr).wait()
        pltpu.make_async_copy(rel_hbm.at[cp_v.at[sl]], rr, sem_gr).wait()

    def ground_compute(ci, bufs):
        e1, e2, rr = bufs

        def grp_body(gi, _):
            for e16 in range(16):
                e = gi * 16 + e16
                part = (e1[e, pl.ds(0, 16)] * rr[e, pl.ds(0, 16)]
                        * e2[e, pl.ds(0, 16)])
                for j in (16, 32, 48):
                    part = part + (e1[e, pl.ds(j, 16)]
                                   * rr[e, pl.ds(j, 16)]
                                   * e2[e, pl.ds(j, 16)])
                t_v[pl.ds(e16 * 16, 16)] = part
            lanes = iota * 16
            acc = plsc.load_gather(t_v, [lanes])
            for c in range(1, 16):
                acc = acc + plsc.load_gather(t_v, [lanes + c])
            eidv = ceid_v[pl.ds(ci * _GC + gi * 16, 16)]
            plsc.store_scatter(g_v, [eidv], acc)
            return 0
        lax.fori_loop(0, _GC // 16, grp_body, 0)

    for r in range(2):
        row = wid * 2 + r

        cnt_g = jnp.int32(0)
        for q in range(_TG // _QC):
            qsl = pl.ds(q * _QC, _QC)
            pltpu.sync_copy(first_hbm.at[0, row, qsl], pq_v)
            pltpu.sync_copy(first_hbm.at[1, row, qsl], a1q_v)
            pltpu.sync_copy(first_hbm.at[2, row, qsl], a2q_v)
            pltpu.sync_copy(mask_hbm.at[row, qsl], mq_v)

            def cbody(i, cnt):
                sl = pl.ds(i * 16, 16)
                pv = pq_v[sl]
                a1v = a1q_v[sl]
                a2v = a2q_v[sl]
                mv = mq_v[sl]
                base = pv * _V
                osl = pl.ds(q * _QC + i * 16, 16)
                tidx_v[osl] = base + a1v
                hidx_v[osl] = base + a2v
                ground = ((mv != 0) & (a1v <= _C_NO) & (a2v <= _C_NO)
                          & (pv != 0))
                csl = pl.ds(cnt, 16)
                plsc.store_compressed(ca1_v.at[csl], a1v, mask=ground)
                plsc.store_compressed(ca2_v.at[csl], a2v, mask=ground)
                plsc.store_compressed(cp_v.at[csl], pv, mask=ground)
                eid = q * _QC + i * 16 + iota
                plsc.store_compressed(ceid_v.at[csl], eid, mask=ground)
                pc = plsc.all_reduce_population_count(ground)
                return cnt + jnp.max(pc)
            cnt_g = lax.fori_loop(0, _QC // 16, cbody, cnt_g)

            for j in range(_QC // _PC):
                sl = pl.ds(q * _QC + j * _PC, _PC)
                pltpu.async_copy(tail_hbm.at[tidx_v.at[sl]], pt_v.at[sl],
                                 sem_pg)
                pltpu.async_copy(head_hbm.at[hidx_v.at[sl]], ph_v.at[sl],
                                 sem_pg)

        def pad_body(j, _):
            sl = pl.ds(cnt_g + j * 16, 16)
            z = jnp.zeros((16,), jnp.int32)
            ca1_v[sl] = z
            ca2_v[sl] = z
            cp_v[sl] = z
            ceid_v[sl] = _TG + iota
            return 0
        lax.fori_loop(0, 2 * _GC // 16, pad_body, 0)

        ncg = (cnt_g + _GC - 1) // _GC
        npairs = jnp.maximum(1, (ncg + 1) // 2)
        A = (e1a_v, e2a_v, rra_v)
        Bb = (e1b_v, e2b_v, rrb_v)
        ground_start(0, A)

        def pair_body(cj, _):
            c0 = cj * 2
            ground_start(c0 + 1, Bb)
            ground_wait(A)
            ground_compute(c0, A)

            @pl.when(cj < npairs - 1)
            def _():
                ground_start(c0 + 2, A)

            ground_wait(Bb)
            ground_compute(c0 + 1, Bb)
            return 0
        lax.fori_loop(0, npairs, pair_body, 0)

        def drain_body(j, _):
            sl = pl.ds(j * _PC, _PC)
            pltpu.make_async_copy(tail_hbm.at[tidx_v.at[sl]], pt_v.at[sl],
                                  sem_pg).wait()
            pltpu.make_async_copy(head_hbm.at[hidx_v.at[sl]], ph_v.at[sl],
                                  sem_pg).wait()
            return 0
        lax.fori_loop(0, _TG // _PC, drain_body, 0)

        pltpu.sync_copy(g_v.at[pl.ds(0, _TG)], g_hbm.at[row])
        pltpu.sync_copy(pt_v, pt_hbm.at[row])
        pltpu.sync_copy(ph_v, ph_hbm.at[row])


_score_call = functools.partial(
    pl.kernel,
    out_type=(
        jax.ShapeDtypeStruct((_B, _TG), jnp.float32),
        jax.ShapeDtypeStruct((_B, _TG), jnp.float32),
        jax.ShapeDtypeStruct((_B, _TG), jnp.float32),
    ),
    mesh=plsc.VectorSubcoreMesh(core_axis_name="c", subcore_axis_name="s"),
    compiler_params=pltpu.CompilerParams(
        needs_layout_passes=False, use_tc_tiling_on_sc=False),
    scratch_types=[
        pltpu.VMEM((_QC,), jnp.int32),
        pltpu.VMEM((_QC,), jnp.int32),
        pltpu.VMEM((_QC,), jnp.int32),
        pltpu.VMEM((_QC,), jnp.int32),
        pltpu.VMEM((_TG,), jnp.int32),
        pltpu.VMEM((_TG,), jnp.int32),
        pltpu.VMEM((_TG + 2 * _GC,), jnp.int32),
        pltpu.VMEM((_TG + 2 * _GC,), jnp.int32),
        pltpu.VMEM((_TG + 2 * _GC,), jnp.int32),
        pltpu.VMEM((_TG + 2 * _GC,), jnp.int32),
        pltpu.VMEM((_TG + 16,), jnp.float32),
        pltpu.VMEM((_TG,), jnp.float32),
        pltpu.VMEM((_TG,), jnp.float32),
        pltpu.VMEM((_GC, _D), jnp.float32),
        pltpu.VMEM((_GC, _D), jnp.float32),
        pltpu.VMEM((_GC, _D), jnp.float32),
        pltpu.VMEM((_GC, _D), jnp.float32),
        pltpu.VMEM((_GC, _D), jnp.float32),
        pltpu.VMEM((_GC, _D), jnp.float32),
        pltpu.VMEM((256,), jnp.float32),
        pltpu.SemaphoreType.DMA,
        pltpu.SemaphoreType.DMA,
        pltpu.SemaphoreType.DMA,
    ],
)(_score_body)


def _select_body(first_ref, mask_ref, g_ref, pt_ref, ph_ref, out_ref):
    p = first_ref[0]
    a1 = first_ref[1]
    a2 = first_ref[2]
    mask = mask_ref[...] != 0
    g = g_ref[...]
    pt = pt_ref[...]
    ph = ph_ref[...]

    is_ground = mask & (a1 <= _C_NO) & (a2 <= _C_NO) & (p != 0)
    tail_case = (a1 > 0) & (a1 <= _C_NO) & (a2 > _C_NO)
    head_case = (a1 > _C_NO) & (a2 > 0) & (a2 <= _C_NO)
    is_partial = mask & (~is_ground) & (p != 0) & (tail_case | head_case)
    p_scores = jnp.where(tail_case, pt, jnp.where(head_case, ph, 0.0))
    scores = jnp.where(is_partial, p_scores, jnp.where(is_ground, g, 0.0))
    scored = is_ground | (is_partial & (p_scores > 0.0))
    uncond = mask & (~is_ground) & (~is_partial)

    topk = jnp.where(scored, scores, -jnp.inf)
    x = lax.bitcast_convert_type(topk, jnp.int32)
    s = x ^ ((x >> 31) & jnp.int32(0x7FFFFFFF))

    def step(i, P):
        b = jnp.int32(31) - i
        Pp = P | (jnp.int32(1) << b)
        v = Pp ^ jnp.int32(_INT_MIN)
        cnt = jnp.sum((s >= v).astype(jnp.int32), axis=1, keepdims=True)
        return jnp.where(cnt >= _TOP_K, Pp, P)

    P = lax.fori_loop(0, 32, step, jnp.zeros((_B, 1), jnp.int32))
    T = P ^ jnp.int32(_INT_MIN)

    gt = s > T
    cnt_gt = jnp.sum(gt.astype(jnp.int32), axis=1, keepdims=True)
    r = _TOP_K - cnt_gt
    eq = s == T
    inc = eq.astype(jnp.int32)
    sh = 1
    while sh < _TG:
        inc = inc + jnp.concatenate(
            [jnp.zeros((_B, sh), jnp.int32), inc[:, :_TG - sh]], axis=1)
        sh *= 2
    keep = gt | (eq & (inc <= r))
    new_mask = mask & (keep | uncond)
    out_ref[...] = new_mask.astype(jnp.int32)


def _select_call(first, mask_i32, g, pt, ph):
    return pl.pallas_call(
        _select_body,
        out_shape=jax.ShapeDtypeStruct((_B, _TG), jnp.int32),
    )(first, mask_i32, g, pt, ph)


def kernel(body, mask, rule_idx, d, ent_emb, rel_emb, max_tail_score, max_head_score):
    first = jnp.transpose(body[:, :, 0, :], (2, 0, 1))
    mask_i32 = mask.astype(jnp.int32)
    tail_flat = max_tail_score.reshape(-1)
    head_flat = max_head_score.reshape(-1)
    g, pt, ph = _score_call(first, mask_i32, ent_emb, rel_emb,
                            tail_flat, head_flat)
    keep = _select_call(first, mask_i32, g, pt, ph)
    new_mask = keep != 0
    return (body, new_mask, rule_idx)


# speedup vs baseline: 1.2273x; 1.0097x over previous
import functools

import jax
import jax.numpy as jnp
from jax import lax
from jax.experimental import pallas as pl
from jax.experimental.pallas import tpu as pltpu
from jax.experimental.pallas import tpu_sc as plsc

_B, _TG = 64, 8192
_V, _D = 4096, 64
_C_NO, _TOP_K = 3500, 1024
_CHUNK = 128
_NCHUNK = _TG // _CHUNK
_NPAIR = _NCHUNK // 2
_INT_MIN = -2147483648


def _score_body(first_hbm, ent_hbm, rel_hbm, tail_hbm, head_hbm,
                g_hbm, pt_hbm, ph_hbm,
                p_v, a1_v, a2_v, tidx_v, hidx_v, g_v, pt_v, ph_v,
                e1a_v, e2a_v, rra_v, e1b_v, e2b_v, rrb_v, t_v,
                sem_pg, sem_gr):
    cid = lax.axis_index("c")
    sid = lax.axis_index("s")
    wid = sid * 2 + cid

    iota = lax.iota(jnp.int32, 16)

    def ground_start(ci, bufs):
        e1, e2, rr = bufs
        sl = pl.ds(ci * _CHUNK, _CHUNK)
        pltpu.async_copy(ent_hbm.at[a1_v.at[sl]], e1, sem_gr)
        pltpu.async_copy(ent_hbm.at[a2_v.at[sl]], e2, sem_gr)
        pltpu.async_copy(rel_hbm.at[p_v.at[sl]], rr, sem_gr)

    def ground_wait(bufs):
        e1, e2, rr = bufs
        sl = pl.ds(0, _CHUNK)
        pltpu.make_async_copy(ent_hbm.at[a1_v.at[sl]], e1, sem_gr).wait()
        pltpu.make_async_copy(ent_hbm.at[a2_v.at[sl]], e2, sem_gr).wait()
        pltpu.make_async_copy(rel_hbm.at[p_v.at[sl]], rr, sem_gr).wait()

    def ground_compute(ci, bufs):
        e1, e2, rr = bufs

        def grp_body(gi, _):
            for e16 in range(16):
                e = gi * 16 + e16
                part = (e1[e, pl.ds(0, 16)] * rr[e, pl.ds(0, 16)]
                        * e2[e, pl.ds(0, 16)])
                for j in (16, 32, 48):
                    part = part + (e1[e, pl.ds(j, 16)]
                                   * rr[e, pl.ds(j, 16)]
                                   * e2[e, pl.ds(j, 16)])
                t_v[pl.ds(e16 * 16, 16)] = part
            lanes = iota * 16
            acc = plsc.load_gather(t_v, [lanes])
            for c in range(1, 16):
                acc = acc + plsc.load_gather(t_v, [lanes + c])
            g_v[pl.ds(ci * _CHUNK + gi * 16, 16)] = acc
            return 0
        lax.fori_loop(0, _CHUNK // 16, grp_body, 0)

    for r in range(2):
        row = wid * 2 + r
        pltpu.sync_copy(first_hbm.at[0, row], p_v)
        pltpu.sync_copy(first_hbm.at[1, row], a1_v)
        pltpu.sync_copy(first_hbm.at[2, row], a2_v)

        def idx_body(i, _):
            sl = pl.ds(i * 16, 16)
            base = p_v[sl] * _V
            tidx_v[sl] = base + a1_v[sl]
            hidx_v[sl] = base + a2_v[sl]
            return 0
        lax.fori_loop(0, _TG // 16, idx_body, 0)

        for j in range(_NCHUNK):
            sl = pl.ds(j * _CHUNK, _CHUNK)
            pltpu.async_copy(tail_hbm.at[tidx_v.at[sl]], pt_v.at[sl], sem_pg)
            pltpu.async_copy(head_hbm.at[hidx_v.at[sl]], ph_v.at[sl], sem_pg)

        A = (e1a_v, e2a_v, rra_v)
        Bb = (e1b_v, e2b_v, rrb_v)
        ground_start(0, A)

        def pair_body(cj, _):
            c0 = cj * 2
            ground_start(c0 + 1, Bb)
            ground_wait(A)
            ground_compute(c0, A)

            @pl.when(cj < _NPAIR - 1)
            def _():
                ground_start(c0 + 2, A)

            ground_wait(Bb)
            ground_compute(c0 + 1, Bb)
            return 0
        lax.fori_loop(0, _NPAIR, pair_body, 0)

        def drain_body(j, _):
            sl = pl.ds(j * _CHUNK, _CHUNK)
            pltpu.make_async_copy(tail_hbm.at[tidx_v.at[sl]], pt_v.at[sl],
                                  sem_pg).wait()
            pltpu.make_async_copy(head_hbm.at[hidx_v.at[sl]], ph_v.at[sl],
                                  sem_pg).wait()
            return 0
        lax.fori_loop(0, _NCHUNK, drain_body, 0)

        pltpu.sync_copy(g_v, g_hbm.at[row])
        pltpu.sync_copy(pt_v, pt_hbm.at[row])
        pltpu.sync_copy(ph_v, ph_hbm.at[row])


_score_call = functools.partial(
    pl.kernel,
    out_type=(
        jax.ShapeDtypeStruct((_B, _TG), jnp.float32),
        jax.ShapeDtypeStruct((_B, _TG), jnp.float32),
        jax.ShapeDtypeStruct((_B, _TG), jnp.float32),
    ),
    mesh=plsc.VectorSubcoreMesh(core_axis_name="c", subcore_axis_name="s"),
    compiler_params=pltpu.CompilerParams(
        needs_layout_passes=False, use_tc_tiling_on_sc=False),
    scratch_types=[
        pltpu.VMEM((_TG,), jnp.int32),
        pltpu.VMEM((_TG,), jnp.int32),
        pltpu.VMEM((_TG,), jnp.int32),
        pltpu.VMEM((_TG,), jnp.int32),
        pltpu.VMEM((_TG,), jnp.int32),
        pltpu.VMEM((_TG,), jnp.float32),
        pltpu.VMEM((_TG,), jnp.float32),
        pltpu.VMEM((_TG,), jnp.float32),
        pltpu.VMEM((_CHUNK, _D), jnp.float32),
        pltpu.VMEM((_CHUNK, _D), jnp.float32),
        pltpu.VMEM((_CHUNK, _D), jnp.float32),
        pltpu.VMEM((_CHUNK, _D), jnp.float32),
        pltpu.VMEM((_CHUNK, _D), jnp.float32),
        pltpu.VMEM((_CHUNK, _D), jnp.float32),
        pltpu.VMEM((256,), jnp.float32),
        pltpu.SemaphoreType.DMA,
        pltpu.SemaphoreType.DMA,
    ],
)(_score_body)


def _select_body(first_ref, mask_ref, g_ref, pt_ref, ph_ref, out_ref):
    p = first_ref[0]
    a1 = first_ref[1]
    a2 = first_ref[2]
    mask = mask_ref[...] != 0
    g = g_ref[...]
    pt = pt_ref[...]
    ph = ph_ref[...]

    is_ground = mask & (a1 <= _C_NO) & (a2 <= _C_NO) & (p != 0)
    tail_case = (a1 > 0) & (a1 <= _C_NO) & (a2 > _C_NO)
    head_case = (a1 > _C_NO) & (a2 > 0) & (a2 <= _C_NO)
    is_partial = mask & (~is_ground) & (p != 0) & (tail_case | head_case)
    p_scores = jnp.where(tail_case, pt, jnp.where(head_case, ph, 0.0))
    scores = jnp.where(is_partial, p_scores, jnp.where(is_ground, g, 0.0))
    scored = is_ground | (is_partial & (p_scores > 0.0))
    uncond = mask & (~is_ground) & (~is_partial)

    topk = jnp.where(scored, scores, -jnp.inf)
    x = lax.bitcast_convert_type(topk, jnp.int32)
    s = x ^ ((x >> 31) & jnp.int32(0x7FFFFFFF))

    def step(i, P):
        b = jnp.int32(31) - i
        Pp = P | (jnp.int32(1) << b)
        v = Pp ^ jnp.int32(_INT_MIN)
        cnt = jnp.sum((s >= v).astype(jnp.int32), axis=1, keepdims=True)
        return jnp.where(cnt >= _TOP_K, Pp, P)

    P = lax.fori_loop(0, 32, step, jnp.zeros((_B, 1), jnp.int32))
    T = P ^ jnp.int32(_INT_MIN)

    gt = s > T
    cnt_gt = jnp.sum(gt.astype(jnp.int32), axis=1, keepdims=True)
    r = _TOP_K - cnt_gt
    eq = s == T
    inc = eq.astype(jnp.int32)
    sh = 1
    while sh < _TG:
        inc = inc + jnp.concatenate(
            [jnp.zeros((_B, sh), jnp.int32), inc[:, :_TG - sh]], axis=1)
        sh *= 2
    keep = gt | (eq & (inc <= r))
    new_mask = mask & (keep | uncond)
    out_ref[...] = new_mask.astype(jnp.int32)


def _select_call(first, mask_i32, g, pt, ph):
    return pl.pallas_call(
        _select_body,
        out_shape=jax.ShapeDtypeStruct((_B, _TG), jnp.int32),
    )(first, mask_i32, g, pt, ph)


def kernel(body, mask, rule_idx, d, ent_emb, rel_emb, max_tail_score, max_head_score):
    first = jnp.transpose(body[:, :, 0, :], (2, 0, 1))
    mask_i32 = mask.astype(jnp.int32)
    tail_flat = max_tail_score.reshape(-1)
    head_flat = max_head_score.reshape(-1)
    g, pt, ph = _score_call(first, ent_emb, rel_emb, tail_flat, head_flat)
    keep = _select_call(first, mask_i32, g, pt, ph)
    new_mask = keep != 0
    return (body, new_mask, rule_idx)


# speedup vs baseline: 1.3078x; 1.0656x over previous
import functools

import jax
import jax.numpy as jnp
from jax import lax
from jax.experimental import pallas as pl
from jax.experimental.pallas import tpu as pltpu
from jax.experimental.pallas import tpu_sc as plsc

_B, _TG = 64, 8192
_V, _D = 4096, 64
_C_NO, _TOP_K = 3500, 1024
_CHUNK = 128
_NCHUNK = _TG // _CHUNK
_NPAIR = _NCHUNK // 2
_INT_MIN = -2147483648


def _score_body(first_hbm, ent_hbm, rel_hbm, tail_hbm, head_hbm,
                g_hbm, pt_hbm, ph_hbm,
                p_v, a1_v, a2_v, tidx_v, hidx_v, g_v, pt_v, ph_v,
                e1a_v, e2a_v, rra_v, e1b_v, e2b_v, rrb_v, t_v,
                sem_pg, sem_gr):
    cid = lax.axis_index("c")
    sid = lax.axis_index("s")
    wid = sid * 2 + cid

    iota = lax.iota(jnp.int32, 16)

    def ground_start(ci, bufs):
        e1, e2, rr = bufs
        sl = pl.ds(ci * _CHUNK, _CHUNK)
        pltpu.async_copy(ent_hbm.at[a1_v.at[sl]], e1, sem_gr)
        pltpu.async_copy(ent_hbm.at[a2_v.at[sl]], e2, sem_gr)
        pltpu.async_copy(rel_hbm.at[p_v.at[sl]], rr, sem_gr)

    def ground_wait(bufs):
        e1, e2, rr = bufs
        sl = pl.ds(0, _CHUNK)
        pltpu.make_async_copy(ent_hbm.at[a1_v.at[sl]], e1, sem_gr).wait()
        pltpu.make_async_copy(ent_hbm.at[a2_v.at[sl]], e2, sem_gr).wait()
        pltpu.make_async_copy(rel_hbm.at[p_v.at[sl]], rr, sem_gr).wait()

    def ground_compute(ci, bufs):
        e1, e2, rr = bufs

        def grp_body(gi, _):
            for e16 in range(16):
                e = gi * 16 + e16
                part = (e1[e, pl.ds(0, 16)] * rr[e, pl.ds(0, 16)]
                        * e2[e, pl.ds(0, 16)])
                for j in (16, 32, 48):
                    part = part + (e1[e, pl.ds(j, 16)]
                                   * rr[e, pl.ds(j, 16)]
                                   * e2[e, pl.ds(j, 16)])
                t_v[pl.ds(e16 * 16, 16)] = part
            lanes = iota * 16
            acc = plsc.load_gather(t_v, [lanes])
            for c in range(1, 16):
                acc = acc + plsc.load_gather(t_v, [lanes + c])
            g_v[pl.ds(ci * _CHUNK + gi * 16, 16)] = acc
            return 0
        lax.fori_loop(0, _CHUNK // 16, grp_body, 0)

    for r in range(2):
        row = wid * 2 + r
        pltpu.sync_copy(first_hbm.at[0, row], p_v)
        pltpu.sync_copy(first_hbm.at[1, row], a1_v)
        pltpu.sync_copy(first_hbm.at[2, row], a2_v)

        def idx_body(i, _):
            sl = pl.ds(i * 16, 16)
            base = p_v[sl] * _V
            tidx_v[sl] = base + a1_v[sl]
            hidx_v[sl] = base + a2_v[sl]
            return 0
        lax.fori_loop(0, _TG // 16, idx_body, 0)

        def pg_start(ci):
            sl = pl.ds(ci * _CHUNK, _CHUNK)
            pltpu.async_copy(tail_hbm.at[tidx_v.at[sl]], pt_v.at[sl], sem_pg)
            pltpu.async_copy(head_hbm.at[hidx_v.at[sl]], ph_v.at[sl], sem_pg)

        def pg_drain(ci):
            sl = pl.ds(ci * _CHUNK, _CHUNK)
            pltpu.make_async_copy(tail_hbm.at[tidx_v.at[sl]], pt_v.at[sl],
                                  sem_pg).wait()
            pltpu.make_async_copy(head_hbm.at[hidx_v.at[sl]], ph_v.at[sl],
                                  sem_pg).wait()

        A = (e1a_v, e2a_v, rra_v)
        Bb = (e1b_v, e2b_v, rrb_v)
        ground_start(0, A)

        def pair_body(cj, _):
            c0 = cj * 2
            ground_start(c0 + 1, Bb)
            pg_start(c0)
            pg_start(c0 + 1)
            ground_wait(A)
            ground_compute(c0, A)

            @pl.when(cj > 0)
            def _():
                pg_drain(c0 - 2)
                pg_drain(c0 - 1)

            @pl.when(cj < _NPAIR - 1)
            def _():
                ground_start(c0 + 2, A)

            ground_wait(Bb)
            ground_compute(c0 + 1, Bb)
            return 0
        lax.fori_loop(0, _NPAIR, pair_body, 0)

        pg_drain(_NCHUNK - 2)
        pg_drain(_NCHUNK - 1)

        pltpu.sync_copy(g_v, g_hbm.at[row])
        pltpu.sync_copy(pt_v, pt_hbm.at[row])
        pltpu.sync_copy(ph_v, ph_hbm.at[row])


_score_call = functools.partial(
    pl.kernel,
    out_type=(
        jax.ShapeDtypeStruct((_B, _TG), jnp.float32),
        jax.ShapeDtypeStruct((_B, _TG), jnp.float32),
        jax.ShapeDtypeStruct((_B, _TG), jnp.float32),
    ),
    mesh=plsc.VectorSubcoreMesh(core_axis_name="c", subcore_axis_name="s"),
    compiler_params=pltpu.CompilerParams(
        needs_layout_passes=False, use_tc_tiling_on_sc=False),
    scratch_types=[
        pltpu.VMEM((_TG,), jnp.int32),
        pltpu.VMEM((_TG,), jnp.int32),
        pltpu.VMEM((_TG,), jnp.int32),
        pltpu.VMEM((_TG,), jnp.int32),
        pltpu.VMEM((_TG,), jnp.int32),
        pltpu.VMEM((_TG,), jnp.float32),
        pltpu.VMEM((_TG,), jnp.float32),
        pltpu.VMEM((_TG,), jnp.float32),
        pltpu.VMEM((_CHUNK, _D), jnp.float32),
        pltpu.VMEM((_CHUNK, _D), jnp.float32),
        pltpu.VMEM((_CHUNK, _D), jnp.float32),
        pltpu.VMEM((_CHUNK, _D), jnp.float32),
        pltpu.VMEM((_CHUNK, _D), jnp.float32),
        pltpu.VMEM((_CHUNK, _D), jnp.float32),
        pltpu.VMEM((256,), jnp.float32),
        pltpu.SemaphoreType.DMA,
        pltpu.SemaphoreType.DMA,
    ],
)(_score_body)


def _select_body(first_ref, mask_ref, g_ref, pt_ref, ph_ref, out_ref):
    p = first_ref[0]
    a1 = first_ref[1]
    a2 = first_ref[2]
    mask = mask_ref[...] != 0
    g = g_ref[...]
    pt = pt_ref[...]
    ph = ph_ref[...]

    is_ground = mask & (a1 <= _C_NO) & (a2 <= _C_NO) & (p != 0)
    tail_case = (a1 > 0) & (a1 <= _C_NO) & (a2 > _C_NO)
    head_case = (a1 > _C_NO) & (a2 > 0) & (a2 <= _C_NO)
    is_partial = mask & (~is_ground) & (p != 0) & (tail_case | head_case)
    p_scores = jnp.where(tail_case, pt, jnp.where(head_case, ph, 0.0))
    scores = jnp.where(is_partial, p_scores, jnp.where(is_ground, g, 0.0))
    scored = is_ground | (is_partial & (p_scores > 0.0))
    uncond = mask & (~is_ground) & (~is_partial)

    topk = jnp.where(scored, scores, -jnp.inf)
    x = lax.bitcast_convert_type(topk, jnp.int32)
    s = x ^ ((x >> 31) & jnp.int32(0x7FFFFFFF))

    def step(i, P):
        b = jnp.int32(31) - i
        Pp = P | (jnp.int32(1) << b)
        v = Pp ^ jnp.int32(_INT_MIN)
        cnt = jnp.sum((s >= v).astype(jnp.int32), axis=1, keepdims=True)
        return jnp.where(cnt >= _TOP_K, Pp, P)

    P = lax.fori_loop(0, 32, step, jnp.zeros((_B, 1), jnp.int32))
    T = P ^ jnp.int32(_INT_MIN)

    gt = s > T
    cnt_gt = jnp.sum(gt.astype(jnp.int32), axis=1, keepdims=True)
    r = _TOP_K - cnt_gt
    eq = s == T
    inc = eq.astype(jnp.int32)
    sh = 1
    while sh < _TG:
        inc = inc + jnp.concatenate(
            [jnp.zeros((_B, sh), jnp.int32), inc[:, :_TG - sh]], axis=1)
        sh *= 2
    keep = gt | (eq & (inc <= r))
    new_mask = mask & (keep | uncond)
    out_ref[...] = new_mask.astype(jnp.int32)


def _select_call(first, mask_i32, g, pt, ph):
    return pl.pallas_call(
        _select_body,
        out_shape=jax.ShapeDtypeStruct((_B, _TG), jnp.int32),
    )(first, mask_i32, g, pt, ph)


def kernel(body, mask, rule_idx, d, ent_emb, rel_emb, max_tail_score, max_head_score):
    first = jnp.transpose(body[:, :, 0, :], (2, 0, 1))
    mask_i32 = mask.astype(jnp.int32)
    tail_flat = max_tail_score.reshape(-1)
    head_flat = max_head_score.reshape(-1)
    g, pt, ph = _score_call(first, ent_emb, rel_emb, tail_flat, head_flat)
    keep = _select_call(first, mask_i32, g, pt, ph)
    new_mask = keep != 0
    return (body, new_mask, rule_idx)
